# initial kernel scaffold (unmeasured)
import jax
import jax.numpy as jnp
from jax import lax
from jax.experimental import pallas as pl
from jax.experimental.pallas import tpu as pltpu

N_DEV = 16
B, SQ, D = 2, 256, 768
DH, SKV = 64, 512
ROWS = B * SQ
CH = ROWS // N_DEV
HOPS = N_DEV - 1


def kernel(x, Wq, Wo, K_ext, V_ext):
    i_out = lax.axis_index("i")
    Ksl = lax.dynamic_slice_in_dim(K_ext, 2 * i_out, 2, axis=2)
    Vsl = lax.dynamic_slice_in_dim(V_ext, 2 * i_out, 2, axis=2)
    x2 = x.reshape(ROWS, D)

    def body(x_ref, wq_ref, wo_ref, k_ref, v_ref, out_ref,
             staging, rs_ssem, rs_rsem, ag_ssem, ag_rsem):
        i = lax.axis_index("i")
        right = jnp.mod(i + 1, N_DEV)

        q = jnp.dot(x_ref[...], wq_ref[...],
                    preferred_element_type=jnp.float32)
        obs = []
        for b in range(B):
            houts = []
            for u in range(2):
                k_u = k_ref[b, :, u, :]
                v_u = v_ref[b, :, u, :]
                for t4 in range(4):
                    t = 4 * u + t4
                    qh = q[b * SQ:(b + 1) * SQ, t * DH:(t + 1) * DH]
                    s = lax.dot_general(
                        qh, k_u, (((1,), (1,)), ((), ())),
                        preferred_element_type=jnp.float32) * 0.125
                    m = jnp.max(s, axis=-1, keepdims=True)
                    p = jnp.exp(s - m)
                    l = jnp.sum(p, axis=-1, keepdims=True)
                    o = jnp.dot(p, v_u,
                                preferred_element_type=jnp.float32) / l
                    houts.append(o)
            obs.append(jnp.concatenate(houts, axis=1))
        attn = jnp.concatenate(obs, axis=0)
        out_ref[...] = jnp.dot(attn, wo_ref[...],
                               preferred_element_type=jnp.float32)

        for h in range(HOPS):
            c_send = jnp.mod(i - h, N_DEV)
            c_recv = jnp.mod(i - h - 1, N_DEV)
            rdma = pltpu.make_async_remote_copy(
                src_ref=out_ref.at[pl.ds(c_send * CH, CH), :],
                dst_ref=staging.at[h],
                send_sem=rs_ssem.at[h],
                recv_sem=rs_rsem.at[h],
                device_id=(right,),
                device_id_type=pl.DeviceIdType.MESH,
            )
            rdma.start()
            rdma.wait()
            r0 = c_recv * CH
            out_ref[pl.ds(r0, CH), :] = (
                out_ref[pl.ds(r0, CH), :] + staging[h])

        for h in range(HOPS):
            c = jnp.mod(i + 1 - h, N_DEV)
            rdma = pltpu.make_async_remote_copy(
                src_ref=out_ref.at[pl.ds(c * CH, CH), :],
                dst_ref=out_ref.at[pl.ds(c * CH, CH), :],
                send_sem=ag_ssem.at[h],
                recv_sem=ag_rsem.at[h],
                device_id=(right,),
                device_id_type=pl.DeviceIdType.MESH,
            )
            rdma.start()
            rdma.wait()

    out = pl.pallas_call(
        body,
        out_shape=jax.ShapeDtypeStruct((ROWS, D), jnp.float32),
        in_specs=[pl.BlockSpec(memory_space=pltpu.VMEM)] * 5,
        out_specs=pl.BlockSpec(memory_space=pltpu.VMEM),
        scratch_shapes=[
            pltpu.VMEM((HOPS, CH, D), jnp.float32),
            pltpu.SemaphoreType.DMA((HOPS,)),
            pltpu.SemaphoreType.DMA((HOPS,)),
            pltpu.SemaphoreType.DMA((HOPS,)),
            pltpu.SemaphoreType.DMA((HOPS,)),
        ],
        compiler_params=pltpu.CompilerParams(collective_id=0),
    )(x2, Wq, Wo, Ksl, Vsl)
    return out.reshape(B, SQ, D)


# baseline (device time: 106380 ns/iter reference)
import jax
import jax.numpy as jnp
from jax import lax
from jax.experimental import pallas as pl
from jax.experimental.pallas import tpu as pltpu

N_DEV = 16
B, SQ, D = 2, 256, 768
DH, SKV = 64, 512
ROWS = B * SQ
CH = ROWS // N_DEV
HOPS = N_DEV - 1


def kernel(x, Wq, Wo, K_ext, V_ext):
    i_out = lax.axis_index("i")
    Ksl = lax.dynamic_slice_in_dim(K_ext, 2 * i_out, 2, axis=2)
    Vsl = lax.dynamic_slice_in_dim(V_ext, 2 * i_out, 2, axis=2)
    x2 = x.reshape(ROWS, D)

    def body(x_ref, wq_ref, wo_ref, k_ref, v_ref, out_ref,
             staging, rs_ssem, rs_rsem, ag_ssem, ag_rsem):
        i = lax.axis_index("i")
        right = jnp.mod(i + 1, N_DEV)

        q = jnp.dot(x_ref[...], wq_ref[...],
                    preferred_element_type=jnp.float32)
        obs = []
        for b in range(B):
            houts = []
            for u in range(2):
                k_u = k_ref[b, :, u, :]
                v_u = v_ref[b, :, u, :]
                for t4 in range(4):
                    t = 4 * u + t4
                    qh = q[b * SQ:(b + 1) * SQ, t * DH:(t + 1) * DH]
                    s = lax.dot_general(
                        qh, k_u, (((1,), (1,)), ((), ())),
                        preferred_element_type=jnp.float32) * 0.125
                    m = jnp.max(s, axis=-1, keepdims=True)
                    p = jnp.exp(s - m)
                    l = jnp.sum(p, axis=-1, keepdims=True)
                    o = jnp.dot(p, v_u,
                                preferred_element_type=jnp.float32) / l
                    houts.append(o)
            obs.append(jnp.concatenate(houts, axis=1))
        attn = jnp.concatenate(obs, axis=0)
        out_ref[...] = jnp.dot(attn, wo_ref[...],
                               preferred_element_type=jnp.float32)

        for h in range(HOPS):
            c_send = jnp.mod(i - h, N_DEV)
            c_recv = jnp.mod(i - h - 1, N_DEV)
            rdma = pltpu.make_async_remote_copy(
                src_ref=out_ref.at[pl.ds(c_send * CH, CH), :],
                dst_ref=staging.at[h],
                send_sem=rs_ssem.at[h],
                recv_sem=rs_rsem.at[h],
                device_id=(right,),
                device_id_type=pl.DeviceIdType.MESH,
            )
            rdma.start()
            rdma.wait()
            r0 = c_recv * CH
            out_ref[pl.ds(r0, CH), :] = (
                out_ref[pl.ds(r0, CH), :] + staging[h])

        for h in range(HOPS):
            c = jnp.mod(i + 1 - h, N_DEV)
            rdma = pltpu.make_async_remote_copy(
                src_ref=out_ref.at[pl.ds(c * CH, CH), :],
                dst_ref=out_ref.at[pl.ds(c * CH, CH), :],
                send_sem=ag_ssem.at[h],
                recv_sem=ag_rsem.at[h],
                device_id=(right,),
                device_id_type=pl.DeviceIdType.MESH,
            )
            rdma.start()
            rdma.wait()

    out = pl.pallas_call(
        body,
        out_shape=jax.ShapeDtypeStruct((ROWS, D), jnp.float32),
        in_specs=[pl.BlockSpec(memory_space=pltpu.VMEM)] * 5,
        out_specs=pl.BlockSpec(memory_space=pltpu.VMEM),
        scratch_shapes=[
            pltpu.VMEM((HOPS, CH, D), jnp.float32),
            pltpu.SemaphoreType.DMA((HOPS,)),
            pltpu.SemaphoreType.DMA((HOPS,)),
            pltpu.SemaphoreType.DMA((HOPS,)),
            pltpu.SemaphoreType.DMA((HOPS,)),
        ],
    )(x2, Wq, Wo, Ksl, Vsl)
    return out.reshape(B, SQ, D)


# device time: 64907 ns/iter; 1.6390x vs baseline; 1.6390x over previous
import jax
import jax.numpy as jnp
from jax import lax
from jax.experimental import pallas as pl
from jax.experimental.pallas import tpu as pltpu

N_DEV = 16
B, SQ, D = 2, 256, 768
DH, SKV = 64, 512
ROWS = B * SQ
CH = ROWS // N_DEV
MASKS = (1, 3, 4, 8)


def kernel(x, Wq, Wo, K_ext, V_ext):
    i_out = lax.axis_index("i")
    Ksl = lax.dynamic_slice_in_dim(K_ext, 2 * i_out, 2, axis=2)
    Vsl = lax.dynamic_slice_in_dim(V_ext, 2 * i_out, 2, axis=2)
    x2 = x.reshape(ROWS, D)

    def body(x_ref, wq_ref, wo_ref, k_ref, v_ref, out_ref,
             st0, st1, st2, st3, rs_ssem, rs_rsem, ag_ssem, ag_rsem):
        i = lax.axis_index("i")

        q = jnp.dot(x_ref[...], wq_ref[...],
                    preferred_element_type=jnp.float32)
        obs = []
        for b in range(B):
            houts = []
            for u in range(2):
                k_u = k_ref[b, :, u, :]
                v_u = v_ref[b, :, u, :]
                for t4 in range(4):
                    t = 4 * u + t4
                    qh = q[b * SQ:(b + 1) * SQ, t * DH:(t + 1) * DH]
                    s = lax.dot_general(
                        qh, k_u, (((1,), (1,)), ((), ())),
                        preferred_element_type=jnp.float32) * 0.125
                    m = jnp.max(s, axis=-1, keepdims=True)
                    p = jnp.exp(s - m)
                    l = jnp.sum(p, axis=-1, keepdims=True)
                    o = jnp.dot(p, v_u,
                                preferred_element_type=jnp.float32) / l
                    houts.append(o)
            obs.append(jnp.concatenate(houts, axis=1))
        attn = jnp.concatenate(obs, axis=0)
        out_ref[...] = jnp.dot(attn, wo_ref[...],
                               preferred_element_type=jnp.float32)

        i0 = i & 1
        i1 = (i >> 1) & 1
        i2 = (i >> 2) & 1
        i3 = (i >> 3) & 1
        fbits = [i0 ^ i1, i1, i2, i3]
        stagings = [st0, st1, st2, st3]

        base = jnp.int32(0)
        keep_bases = []
        for s in range(4):
            half = 8 >> s
            rh = half * CH
            fb = fbits[s]
            partner = i ^ MASKS[s]
            send_base = base + (1 - fb) * half
            keep_base = base + fb * half
            rdma = pltpu.make_async_remote_copy(
                src_ref=out_ref.at[pl.ds(send_base * CH, rh), :],
                dst_ref=stagings[s],
                send_sem=rs_ssem.at[s],
                recv_sem=rs_rsem.at[s],
                device_id=(partner,),
                device_id_type=pl.DeviceIdType.MESH,
            )
            rdma.start()
            rdma.wait()
            r0 = keep_base * CH
            out_ref[pl.ds(r0, rh), :] = (
                out_ref[pl.ds(r0, rh), :] + stagings[s][...])
            keep_bases.append(keep_base)
            base = keep_base

        for s in (3, 2, 1, 0):
            half = 8 >> s
            rh = half * CH
            partner = i ^ MASKS[s]
            b0 = keep_bases[s] * CH
            rdma = pltpu.make_async_remote_copy(
                src_ref=out_ref.at[pl.ds(b0, rh), :],
                dst_ref=out_ref.at[pl.ds(b0, rh), :],
                send_sem=ag_ssem.at[s],
                recv_sem=ag_rsem.at[s],
                device_id=(partner,),
                device_id_type=pl.DeviceIdType.MESH,
            )
            rdma.start()
            rdma.wait()

    out = pl.pallas_call(
        body,
        out_shape=jax.ShapeDtypeStruct((ROWS, D), jnp.float32),
        in_specs=[pl.BlockSpec(memory_space=pltpu.VMEM)] * 5,
        out_specs=pl.BlockSpec(memory_space=pltpu.VMEM),
        scratch_shapes=[
            pltpu.VMEM((8 * CH, D), jnp.float32),
            pltpu.VMEM((4 * CH, D), jnp.float32),
            pltpu.VMEM((2 * CH, D), jnp.float32),
            pltpu.VMEM((1 * CH, D), jnp.float32),
            pltpu.SemaphoreType.DMA((4,)),
            pltpu.SemaphoreType.DMA((4,)),
            pltpu.SemaphoreType.DMA((4,)),
            pltpu.SemaphoreType.DMA((4,)),
        ],
    )(x2, Wq, Wo, Ksl, Vsl)
    return out.reshape(B, SQ, D)


# device time: 45637 ns/iter; 2.3310x vs baseline; 1.4222x over previous
import jax
import jax.numpy as jnp
from jax import lax
from jax.experimental import pallas as pl
from jax.experimental.pallas import tpu as pltpu

N_DEV = 16
B, SQ, D = 2, 256, 768
DH, SKV = 64, 512
ROWS = B * SQ
CH = ROWS // N_DEV
MASKS = (1, 3, 4, 8)


def kernel(x, Wq, Wo, K_ext, V_ext):
    i_out = lax.axis_index("i")
    Ksl = lax.dynamic_slice_in_dim(K_ext, 2 * i_out, 2, axis=2)
    Vsl = lax.dynamic_slice_in_dim(V_ext, 2 * i_out, 2, axis=2)
    x2 = x.reshape(ROWS, D)

    def body(x_ref, wq_ref, wo_ref, k_ref, v_ref, out_ref,
             sb0, sb1, sb2, sb3, st0, st1, st2, st3, agbuf,
             rs_ssem, rs_rsem, ag_ssem, ag_rsem):
        i = lax.axis_index("i")
        i0 = i & 1
        i1 = (i >> 1) & 1
        i2 = (i >> 2) & 1
        i3 = (i >> 3) & 1
        fbits = [i0 ^ i1, i1, i2, i3]
        send_bufs = [sb0, sb1, sb2, sb3]
        stagings = [st0, st1, st2, st3]

        def store_partial(b):
            qb = jnp.dot(x_ref[pl.ds(b * SQ, SQ), :], wq_ref[...],
                         preferred_element_type=jnp.float32)
            houts = []
            for u in range(2):
                k_u = k_ref[b, :, u, :]
                v_u = v_ref[b, :, u, :]
                for t4 in range(4):
                    t = 4 * u + t4
                    qh = qb[:, t * DH:(t + 1) * DH]
                    s = lax.dot_general(
                        qh, k_u, (((1,), (1,)), ((), ())),
                        preferred_element_type=jnp.float32) * 0.125
                    m = jnp.max(s, axis=-1, keepdims=True)
                    p = jnp.exp(s - m)
                    l = jnp.sum(p, axis=-1, keepdims=True)
                    o = jnp.dot(p, v_u,
                                preferred_element_type=jnp.float32) / l
                    houts.append(o)
            attn = jnp.concatenate(houts, axis=1)
            out_ref[pl.ds(b * SQ, SQ), :] = jnp.dot(
                attn, wo_ref[...], preferred_element_type=jnp.float32)

        fb0 = fbits[0]
        send_b0 = 1 - fb0

        @pl.when(fb0 == 0)
        def _():
            store_partial(1)

        @pl.when(fb0 == 1)
        def _():
            store_partial(0)

        sb0[...] = out_ref[pl.ds(send_b0 * SQ, SQ), :].astype(jnp.bfloat16)
        rdma0 = pltpu.make_async_remote_copy(
            src_ref=sb0,
            dst_ref=st0,
            send_sem=rs_ssem.at[0],
            recv_sem=rs_rsem.at[0],
            device_id=(i ^ MASKS[0],),
            device_id_type=pl.DeviceIdType.MESH,
        )
        rdma0.start()

        @pl.when(fb0 == 0)
        def _():
            store_partial(0)

        @pl.when(fb0 == 1)
        def _():
            store_partial(1)

        rdma0.wait()
        keep_base = fb0 * 8
        out_ref[pl.ds(keep_base * CH, 8 * CH), :] = (
            out_ref[pl.ds(keep_base * CH, 8 * CH), :]
            + st0[...].astype(jnp.float32))
        base = keep_base
        keep_bases = [keep_base]

        for s in range(1, 4):
            half = 8 >> s
            rh = half * CH
            fb = fbits[s]
            partner = i ^ MASKS[s]
            send_base = base + (1 - fb) * half
            keep_base = base + fb * half
            send_bufs[s][...] = out_ref[
                pl.ds(send_base * CH, rh), :].astype(jnp.bfloat16)
            rdma = pltpu.make_async_remote_copy(
                src_ref=send_bufs[s],
                dst_ref=stagings[s],
                send_sem=rs_ssem.at[s],
                recv_sem=rs_rsem.at[s],
                device_id=(partner,),
                device_id_type=pl.DeviceIdType.MESH,
            )
            rdma.start()
            rdma.wait()
            r0 = keep_base * CH
            out_ref[pl.ds(r0, rh), :] = (
                out_ref[pl.ds(r0, rh), :]
                + stagings[s][...].astype(jnp.float32))
            keep_bases.append(keep_base)
            base = keep_base

        agbuf[pl.ds(base * CH, CH), :] = out_ref[
            pl.ds(base * CH, CH), :].astype(jnp.bfloat16)
        for s in (3, 2, 1, 0):
            half = 8 >> s
            rh = half * CH
            partner = i ^ MASKS[s]
            b0 = keep_bases[s] * CH
            rdma = pltpu.make_async_remote_copy(
                src_ref=agbuf.at[pl.ds(b0, rh), :],
                dst_ref=agbuf.at[pl.ds(b0, rh), :],
                send_sem=ag_ssem.at[s],
                recv_sem=ag_rsem.at[s],
                device_id=(partner,),
                device_id_type=pl.DeviceIdType.MESH,
            )
            rdma.start()
            rdma.wait()

        out_ref[...] = agbuf[...].astype(jnp.float32)

    out = pl.pallas_call(
        body,
        out_shape=jax.ShapeDtypeStruct((ROWS, D), jnp.float32),
        in_specs=[pl.BlockSpec(memory_space=pltpu.VMEM)] * 5,
        out_specs=pl.BlockSpec(memory_space=pltpu.VMEM),
        scratch_shapes=[
            pltpu.VMEM((8 * CH, D), jnp.bfloat16),
            pltpu.VMEM((4 * CH, D), jnp.bfloat16),
            pltpu.VMEM((2 * CH, D), jnp.bfloat16),
            pltpu.VMEM((1 * CH, D), jnp.bfloat16),
            pltpu.VMEM((8 * CH, D), jnp.bfloat16),
            pltpu.VMEM((4 * CH, D), jnp.bfloat16),
            pltpu.VMEM((2 * CH, D), jnp.bfloat16),
            pltpu.VMEM((1 * CH, D), jnp.bfloat16),
            pltpu.VMEM((ROWS, D), jnp.bfloat16),
            pltpu.SemaphoreType.DMA((4,)),
            pltpu.SemaphoreType.DMA((4,)),
            pltpu.SemaphoreType.DMA((4,)),
            pltpu.SemaphoreType.DMA((4,)),
        ],
    )(x2, Wq, Wo, Ksl, Vsl)
    return out.reshape(B, SQ, D)


# device time: 39969 ns/iter; 2.6616x vs baseline; 1.1418x over previous
import jax
import jax.numpy as jnp
from jax import lax
from jax.experimental import pallas as pl
from jax.experimental.pallas import tpu as pltpu

N_DEV = 16
B, SQ, D = 2, 256, 768
DH, SKV = 64, 512
ROWS = B * SQ
CH = ROWS // N_DEV


def kernel(x, Wq, Wo, K_ext, V_ext):
    i_out = lax.axis_index("i")
    Ksl = lax.dynamic_slice_in_dim(K_ext, 2 * i_out, 2, axis=2)
    Vsl = lax.dynamic_slice_in_dim(V_ext, 2 * i_out, 2, axis=2)
    x2 = x.reshape(ROWS, D)

    def body(x_ref, wq_ref, wo_ref, k_ref, v_ref, out_ref,
             pbuf, staging, agbuf, rs_ssem, rs_rsem, ag_ssem, ag_rsem):
        i = lax.axis_index("i")
        i3 = (i >> 3) & 1

        def store_partial(b):
            qb = jnp.dot(x_ref[pl.ds(b * SQ, SQ), :], wq_ref[...],
                         preferred_element_type=jnp.float32)
            houts = []
            for u in range(2):
                k_u = k_ref[b, :, u, :]
                v_u = v_ref[b, :, u, :]
                for t4 in range(4):
                    t = 4 * u + t4
                    qh = qb[:, t * DH:(t + 1) * DH]
                    s = lax.dot_general(
                        qh, k_u, (((1,), (1,)), ((), ())),
                        preferred_element_type=jnp.float32) * 0.125
                    m = jnp.max(s, axis=-1, keepdims=True)
                    p = jnp.exp(s - m)
                    l = jnp.sum(p, axis=-1, keepdims=True)
                    o = jnp.dot(p, v_u,
                                preferred_element_type=jnp.float32) / l
                    houts.append(o)
            attn = jnp.concatenate(houts, axis=1)
            out_ref[pl.ds(b * SQ, SQ), :] = jnp.dot(
                attn, wo_ref[...], preferred_element_type=jnp.float32)

        def rs_rdma(k):
            p = i ^ k
            return pltpu.make_async_remote_copy(
                src_ref=pbuf.at[pl.ds(p * CH, CH), :],
                dst_ref=staging.at[k],
                send_sem=rs_ssem.at[k],
                recv_sem=rs_rsem.at[k],
                device_id=(p,),
                device_id_type=pl.DeviceIdType.MESH,
            )

        def ag_rdma(k, dst_rows):
            return pltpu.make_async_remote_copy(
                src_ref=agbuf.at[pl.ds(i * CH, CH), :],
                dst_ref=agbuf.at[pl.ds(dst_rows, CH), :],
                send_sem=ag_ssem.at[k],
                recv_sem=ag_rsem.at[k],
                device_id=(i ^ k,),
                device_id_type=pl.DeviceIdType.MESH,
            )

        store_partial(0)
        pbuf[pl.ds(0, SQ), :] = out_ref[pl.ds(0, SQ), :].astype(jnp.bfloat16)
        for k in range(1, N_DEV):
            @pl.when(((k >> 3) & 1) == i3)
            def _(k=k):
                rs_rdma(k).start()

        store_partial(1)
        pbuf[pl.ds(SQ, SQ), :] = out_ref[pl.ds(SQ, SQ), :].astype(jnp.bfloat16)
        for k in range(1, N_DEV):
            @pl.when(((k >> 3) & 1) != i3)
            def _(k=k):
                rs_rdma(k).start()

        for k in range(1, N_DEV):
            rs_rdma(k).wait_recv()
        red = (out_ref[pl.ds(i * CH, CH), :]
               + staging[1:N_DEV].astype(jnp.float32).sum(axis=0))

        agbuf[pl.ds(i * CH, CH), :] = red.astype(jnp.bfloat16)
        for k in range(1, N_DEV):
            ag_rdma(k, i * CH).start()
        for k in range(1, N_DEV):
            ag_rdma(k, (i ^ k) * CH).wait_recv()
        out_ref[...] = agbuf[...].astype(jnp.float32)

        for k in range(1, N_DEV):
            rs_rdma(k).wait_send()
            ag_rdma(k, i * CH).wait_send()

    out = pl.pallas_call(
        body,
        out_shape=jax.ShapeDtypeStruct((ROWS, D), jnp.float32),
        in_specs=[pl.BlockSpec(memory_space=pltpu.VMEM)] * 5,
        out_specs=pl.BlockSpec(memory_space=pltpu.VMEM),
        scratch_shapes=[
            pltpu.VMEM((ROWS, D), jnp.bfloat16),
            pltpu.VMEM((N_DEV, CH, D), jnp.bfloat16),
            pltpu.VMEM((ROWS, D), jnp.bfloat16),
            pltpu.SemaphoreType.DMA((N_DEV,)),
            pltpu.SemaphoreType.DMA((N_DEV,)),
            pltpu.SemaphoreType.DMA((N_DEV,)),
            pltpu.SemaphoreType.DMA((N_DEV,)),
        ],
    )(x2, Wq, Wo, Ksl, Vsl)
    return out.reshape(B, SQ, D)
